# baseline (speedup 1.0000x reference)
import functools

import jax
import jax.numpy as jnp
from jax import lax
from jax.experimental import pallas as pl
from jax.experimental.pallas import tpu as pltpu
from jax.experimental.pallas import tpu_sc as plsc

NC = 2
NS = 16
NW = NC * NS
CH = 128


def _gather_body(idx_hbm, table_hbm, out_hbm, idx_v, buf0, buf1, sem0, sem1,
                 *, n_chunk):
    wid = lax.axis_index("s") * NC + lax.axis_index("c")
    rows_w = n_chunk * CH
    base = wid * rows_w
    pltpu.sync_copy(idx_hbm.at[pl.ds(base, rows_w)], idx_v)

    bufs = (buf0, buf1)
    sems = (sem0, sem1)

    def start(j):
        return pltpu.async_copy(
            table_hbm.at[idx_v.at[pl.ds(j * CH, CH)]], bufs[j % 2], sems[j % 2])

    descs = [None] * n_chunk
    descs[0] = start(0)
    for j in range(n_chunk):
        if j + 1 < n_chunk:
            descs[j + 1] = start(j + 1)
        descs[j].wait()
        pltpu.sync_copy(bufs[j % 2], out_hbm.at[pl.ds(base + j * CH, CH)])


def _sc_gather(idx, table2, n_rows):
    n_chunk = n_rows // (NW * CH)
    mesh = plsc.VectorSubcoreMesh(core_axis_name="c", subcore_axis_name="s")
    body = functools.partial(_gather_body, n_chunk=n_chunk)
    return pl.kernel(
        body,
        out_type=jax.ShapeDtypeStruct((n_rows, 128), jnp.float32),
        mesh=mesh,
        scratch_types=[
            pltpu.VMEM((n_rows // NW,), jnp.int32),
            pltpu.VMEM((CH, 128), jnp.float32),
            pltpu.VMEM((CH, 128), jnp.float32),
            pltpu.SemaphoreType.DMA,
            pltpu.SemaphoreType.DMA,
        ],
    )(idx, table2)


def _mlp_body(x3_ref, par_ref, w13_ref, b1_ref, w2_ref, b2_ref, out_ref,
              *, seq, hidden):
    blk = out_ref.shape[0]
    acc = jnp.zeros((blk, hidden), jnp.float32) + b1_ref[...]
    for p in range(seq):
        xp = x3_ref[p]
        parp = par_ref[p].reshape(blk, 1)
        left = xp[:, :64]
        right = xp[:, 64:]
        sel = left + parp * (right - left)
        acc += jnp.dot(sel, w13_ref[p], preferred_element_type=jnp.float32)
    h = jnp.maximum(acc, 0.0)
    logits = jnp.dot(h, w2_ref[...],
                     preferred_element_type=jnp.float32) + b2_ref[...]
    m = jnp.max(logits, axis=1, keepdims=True)
    e = logits - m
    lse = jnp.log(jnp.sum(jnp.exp(e), axis=1, keepdims=True))
    out_ref[...] = e - lse


def _tc_mlp(x3, par, w1, b1, w2, b2, num_tags):
    seq, bs, _ = x3.shape
    in_dim, hidden = w1.shape
    emb = in_dim // seq
    blk = 2048
    grid = bs // blk
    body = functools.partial(_mlp_body, seq=seq, hidden=hidden)
    return pl.pallas_call(
        body,
        grid=(grid,),
        in_specs=[
            pl.BlockSpec((seq, blk, 128), lambda i: (0, i, 0)),
            pl.BlockSpec((seq, blk), lambda i: (0, i)),
            pl.BlockSpec((seq, emb, hidden), lambda i: (0, 0, 0)),
            pl.BlockSpec((1, hidden), lambda i: (0, 0)),
            pl.BlockSpec((hidden, num_tags), lambda i: (0, 0)),
            pl.BlockSpec((1, num_tags), lambda i: (0, 0)),
        ],
        out_specs=pl.BlockSpec((blk, num_tags), lambda i: (i, 0)),
        out_shape=jax.ShapeDtypeStruct((bs, num_tags), jnp.float32),
    )(x3, par, w1.reshape(seq, emb, hidden), b1.reshape(1, hidden), w2,
      b2.reshape(1, num_tags))


def kernel(Xtoks_IDs, emb_table, W1, b1, W2, b2):
    bs, seq = Xtoks_IDs.shape
    vocab, emb = emb_table.shape
    num_tags = W2.shape[1]

    toks_t = Xtoks_IDs.astype(jnp.int32).T
    pair_idx = (toks_t >> 1).reshape(-1)
    par = (toks_t & 1).astype(jnp.float32)

    table2 = emb_table.reshape(vocab // 2, 2 * emb)
    rows = _sc_gather(pair_idx, table2, bs * seq)
    x3 = rows.reshape(seq, bs, 2 * emb)
    return _tc_mlp(x3, par, W1, b1, W2, b2, num_tags)


# speedup vs baseline: 1.2646x; 1.2646x over previous
import functools

import jax
import jax.numpy as jnp
from jax import lax
from jax.experimental import pallas as pl
from jax.experimental.pallas import tpu as pltpu
from jax.experimental.pallas import tpu_sc as plsc

NC = 2
NS = 16
NW = NC * NS
CH = 128
BV = 2048


def _reformat_body(xt_ref, out_ref):
    x = xt_ref[...]
    xt = x.T
    for g in range(BV // 256):
        lo = xt[g * 256:g * 256 + 128]
        hi = xt[g * 256 + 128:g * 256 + 256]
        out_ref[g * 128:(g + 1) * 128, :] = jnp.concatenate([lo, hi], axis=1)


def _tc_reformat(table_t):
    emb, vocab = table_t.shape
    grid = pl.cdiv(vocab, BV)
    return pl.pallas_call(
        _reformat_body,
        grid=(grid,),
        in_specs=[pl.BlockSpec((emb, BV), lambda i: (0, i))],
        out_specs=pl.BlockSpec((BV // 2, 2 * emb), lambda i: (i, 0)),
        out_shape=jax.ShapeDtypeStruct((grid * BV // 2, 2 * emb), jnp.float32),
    )(table_t)


def _gather_body(idx_hbm, table_hbm, out_hbm, idx_v, buf0, buf1, sem0, sem1,
                 *, n_chunk):
    wid = lax.axis_index("s") * NC + lax.axis_index("c")
    rows_w = n_chunk * CH
    base = wid * rows_w
    pltpu.sync_copy(idx_hbm.at[pl.ds(base, rows_w)], idx_v)

    bufs = (buf0, buf1)
    sems = (sem0, sem1)

    def start(j):
        return pltpu.async_copy(
            table_hbm.at[idx_v.at[pl.ds(j * CH, CH)]], bufs[j % 2], sems[j % 2])

    descs = [None] * n_chunk
    descs[0] = start(0)
    for j in range(n_chunk):
        if j + 1 < n_chunk:
            descs[j + 1] = start(j + 1)
        descs[j].wait()
        pltpu.sync_copy(bufs[j % 2], out_hbm.at[pl.ds(base + j * CH, CH)])


def _sc_gather(idx, table2, n_rows):
    n_chunk = n_rows // (NW * CH)
    mesh = plsc.VectorSubcoreMesh(core_axis_name="c", subcore_axis_name="s")
    body = functools.partial(_gather_body, n_chunk=n_chunk)
    return pl.kernel(
        body,
        out_type=jax.ShapeDtypeStruct((n_rows, 128), jnp.float32),
        mesh=mesh,
        scratch_types=[
            pltpu.VMEM((n_rows // NW,), jnp.int32),
            pltpu.VMEM((CH, 128), jnp.float32),
            pltpu.VMEM((CH, 128), jnp.float32),
            pltpu.SemaphoreType.DMA,
            pltpu.SemaphoreType.DMA,
        ],
    )(idx, table2)


def _mlp_body(x3_ref, par_ref, w13_ref, b1_ref, w2_ref, b2_ref, out_ref,
              *, seq, hidden):
    blk = out_ref.shape[0]
    acc = jnp.zeros((blk, hidden), jnp.float32) + b1_ref[...]
    for p in range(seq):
        xp = x3_ref[p]
        parp = par_ref[p].reshape(blk, 1)
        left = xp[:, :64]
        right = xp[:, 64:]
        sel = left + parp * (right - left)
        acc += jnp.dot(sel, w13_ref[p], preferred_element_type=jnp.float32)
    h = jnp.maximum(acc, 0.0)
    logits = jnp.dot(h, w2_ref[...],
                     preferred_element_type=jnp.float32) + b2_ref[...]
    m = jnp.max(logits, axis=1, keepdims=True)
    e = logits - m
    lse = jnp.log(jnp.sum(jnp.exp(e), axis=1, keepdims=True))
    out_ref[...] = e - lse


def _tc_mlp(x3, par, w1, b1, w2, b2, num_tags):
    seq, bs, _ = x3.shape
    in_dim, hidden = w1.shape
    emb = in_dim // seq
    blk = 2048
    grid = bs // blk
    body = functools.partial(_mlp_body, seq=seq, hidden=hidden)
    return pl.pallas_call(
        body,
        grid=(grid,),
        in_specs=[
            pl.BlockSpec((seq, blk, 128), lambda i: (0, i, 0)),
            pl.BlockSpec((seq, blk), lambda i: (0, i)),
            pl.BlockSpec((seq, emb, hidden), lambda i: (0, 0, 0)),
            pl.BlockSpec((1, hidden), lambda i: (0, 0)),
            pl.BlockSpec((hidden, num_tags), lambda i: (0, 0)),
            pl.BlockSpec((1, num_tags), lambda i: (0, 0)),
        ],
        out_specs=pl.BlockSpec((blk, num_tags), lambda i: (i, 0)),
        out_shape=jax.ShapeDtypeStruct((bs, num_tags), jnp.float32),
    )(x3, par, w1.reshape(seq, emb, hidden), b1.reshape(1, hidden), w2,
      b2.reshape(1, num_tags))


def kernel(Xtoks_IDs, emb_table, W1, b1, W2, b2):
    bs, seq = Xtoks_IDs.shape
    vocab, emb = emb_table.shape
    num_tags = W2.shape[1]

    toks_t = Xtoks_IDs.astype(jnp.int32).T
    pair_idx = (((toks_t >> 8) << 7) | (toks_t & 127)).reshape(-1)
    par = ((toks_t >> 7) & 1).astype(jnp.float32)

    table2 = _tc_reformat(emb_table.T)
    rows = _sc_gather(pair_idx, table2, bs * seq)
    x3 = rows.reshape(seq, bs, 2 * emb)
    return _tc_mlp(x3, par, W1, b1, W2, b2, num_tags)


# speedup vs baseline: 2.3571x; 1.8639x over previous
import functools

import jax
import jax.numpy as jnp
from jax import lax
from jax.experimental import pallas as pl
from jax.experimental.pallas import tpu as pltpu
from jax.experimental.pallas import tpu_sc as plsc

NC = 2
NS = 16
NW = NC * NS
CH = 128
BV = 8192


def _reformat_body(xt_ref, out_ref):
    x = xt_ref[...]
    z = jnp.concatenate([x[:, :BV // 2], x[:, BV // 2:]], axis=0)
    out_ref[...] = z.T


def _tc_reformat(table_t):
    emb, vocab = table_t.shape
    grid = pl.cdiv(vocab, BV)
    return pl.pallas_call(
        _reformat_body,
        grid=(grid,),
        in_specs=[pl.BlockSpec((emb, BV), lambda i: (0, i))],
        out_specs=pl.BlockSpec((BV // 2, 2 * emb), lambda i: (i, 0)),
        out_shape=jax.ShapeDtypeStruct((grid * BV // 2, 2 * emb), jnp.float32),
    )(table_t)


def _gather_body(idx_hbm, table_hbm, out_hbm, idx_v, buf0, buf1, sem0, sem1,
                 *, n_chunk):
    wid = lax.axis_index("s") * NC + lax.axis_index("c")
    rows_w = n_chunk * CH
    base = wid * rows_w
    pltpu.sync_copy(idx_hbm.at[pl.ds(base, rows_w)], idx_v)

    bufs = (buf0, buf1)
    sems = (sem0, sem1)

    def start(j):
        return pltpu.async_copy(
            table_hbm.at[idx_v.at[pl.ds(j * CH, CH)]], bufs[j % 2], sems[j % 2])

    descs = [None] * n_chunk
    descs[0] = start(0)
    for j in range(n_chunk):
        if j + 1 < n_chunk:
            descs[j + 1] = start(j + 1)
        descs[j].wait()
        pltpu.sync_copy(bufs[j % 2], out_hbm.at[pl.ds(base + j * CH, CH)])


def _sc_gather(idx, table2, n_rows):
    n_chunk = n_rows // (NW * CH)
    mesh = plsc.VectorSubcoreMesh(core_axis_name="c", subcore_axis_name="s")
    body = functools.partial(_gather_body, n_chunk=n_chunk)
    return pl.kernel(
        body,
        out_type=jax.ShapeDtypeStruct((n_rows, 128), jnp.float32),
        mesh=mesh,
        scratch_types=[
            pltpu.VMEM((n_rows // NW,), jnp.int32),
            pltpu.VMEM((CH, 128), jnp.float32),
            pltpu.VMEM((CH, 128), jnp.float32),
            pltpu.SemaphoreType.DMA,
            pltpu.SemaphoreType.DMA,
        ],
    )(idx, table2)


def _mlp_body(x3_ref, par_ref, w13_ref, b1_ref, w2_ref, b2_ref, out_ref,
              *, seq, hidden):
    blk = out_ref.shape[0]
    acc = jnp.zeros((blk, hidden), jnp.float32) + b1_ref[...]
    for p in range(seq):
        xp = x3_ref[p]
        parp = par_ref[p].reshape(blk, 1)
        left = xp[:, :64]
        right = xp[:, 64:]
        sel = left + parp * (right - left)
        acc += jnp.dot(sel, w13_ref[p], preferred_element_type=jnp.float32)
    h = jnp.maximum(acc, 0.0)
    logits = jnp.dot(h, w2_ref[...],
                     preferred_element_type=jnp.float32) + b2_ref[...]
    m = jnp.max(logits, axis=1, keepdims=True)
    e = logits - m
    lse = jnp.log(jnp.sum(jnp.exp(e), axis=1, keepdims=True))
    out_ref[...] = e - lse


def _tc_mlp(x3, par, w1, b1, w2, b2, num_tags):
    seq, bs, _ = x3.shape
    in_dim, hidden = w1.shape
    emb = in_dim // seq
    blk = 2048
    grid = bs // blk
    body = functools.partial(_mlp_body, seq=seq, hidden=hidden)
    return pl.pallas_call(
        body,
        grid=(grid,),
        in_specs=[
            pl.BlockSpec((seq, blk, 128), lambda i: (0, i, 0)),
            pl.BlockSpec((seq, blk), lambda i: (0, i)),
            pl.BlockSpec((seq, emb, hidden), lambda i: (0, 0, 0)),
            pl.BlockSpec((1, hidden), lambda i: (0, 0)),
            pl.BlockSpec((hidden, num_tags), lambda i: (0, 0)),
            pl.BlockSpec((1, num_tags), lambda i: (0, 0)),
        ],
        out_specs=pl.BlockSpec((blk, num_tags), lambda i: (i, 0)),
        out_shape=jax.ShapeDtypeStruct((bs, num_tags), jnp.float32),
    )(x3, par, w1.reshape(seq, emb, hidden), b1.reshape(1, hidden), w2,
      b2.reshape(1, num_tags))


def kernel(Xtoks_IDs, emb_table, W1, b1, W2, b2):
    bs, seq = Xtoks_IDs.shape
    vocab, emb = emb_table.shape
    num_tags = W2.shape[1]

    toks_t = Xtoks_IDs.astype(jnp.int32).T
    blk_i = toks_t // BV
    loc = toks_t % BV
    pair_idx = (blk_i * (BV // 2) + loc % (BV // 2)).reshape(-1)
    par = (loc // (BV // 2)).astype(jnp.float32)

    table2 = _tc_reformat(emb_table.T)
    rows = _sc_gather(pair_idx, table2, bs * seq)
    x3 = rows.reshape(seq, bs, 2 * emb)
    return _tc_mlp(x3, par, W1, b1, W2, b2, num_tags)


# speedup vs baseline: 2.6866x; 1.1398x over previous
import functools

import jax
import jax.numpy as jnp
from jax import lax
from jax.experimental import pallas as pl
from jax.experimental.pallas import tpu as pltpu
from jax.experimental.pallas import tpu_sc as plsc

NC = 2
NS = 16
NW = NC * NS
CH = 128
BV = 16384


def _reformat_body(xt_ref, out_ref):
    x = xt_ref[...]
    z = jnp.concatenate([x[:, :BV // 2], x[:, BV // 2:]], axis=0)
    out_ref[...] = z.T


def _tc_reformat(table_t):
    emb, vocab = table_t.shape
    grid = pl.cdiv(vocab, BV)
    return pl.pallas_call(
        _reformat_body,
        grid=(grid,),
        in_specs=[pl.BlockSpec((emb, BV), lambda i: (0, i))],
        out_specs=pl.BlockSpec((BV // 2, 2 * emb), lambda i: (i, 0)),
        out_shape=jax.ShapeDtypeStruct((grid * BV // 2, 2 * emb), jnp.float32),
    )(table_t)


def _gather_body(idx_hbm, table_hbm, out_hbm, idx_v, buf0, buf1, sem0, sem1,
                 *, n_chunk):
    wid = lax.axis_index("s") * NC + lax.axis_index("c")
    rows_w = n_chunk * CH
    base = wid * rows_w
    pltpu.sync_copy(idx_hbm.at[pl.ds(base, rows_w)], idx_v)

    bufs = (buf0, buf1)
    sems = (sem0, sem1)

    def start(j):
        return pltpu.async_copy(
            table_hbm.at[idx_v.at[pl.ds(j * CH, CH)]], bufs[j % 2], sems[j % 2])

    descs = [None] * n_chunk
    descs[0] = start(0)
    for j in range(n_chunk):
        if j + 1 < n_chunk:
            descs[j + 1] = start(j + 1)
        descs[j].wait()
        pltpu.sync_copy(bufs[j % 2], out_hbm.at[pl.ds(base + j * CH, CH)])


def _sc_gather(idx, table2, n_rows):
    n_chunk = n_rows // (NW * CH)
    mesh = plsc.VectorSubcoreMesh(core_axis_name="c", subcore_axis_name="s")
    body = functools.partial(_gather_body, n_chunk=n_chunk)
    return pl.kernel(
        body,
        out_type=jax.ShapeDtypeStruct((n_rows, 128), jnp.float32),
        mesh=mesh,
        scratch_types=[
            pltpu.VMEM((n_rows // NW,), jnp.int32),
            pltpu.VMEM((CH, 128), jnp.float32),
            pltpu.VMEM((CH, 128), jnp.float32),
            pltpu.SemaphoreType.DMA,
            pltpu.SemaphoreType.DMA,
        ],
    )(idx, table2)


def _mlp_body(x3_ref, par_ref, w13_ref, b1_ref, w2_ref, b2_ref, out_ref,
              *, seq, hidden):
    blk = out_ref.shape[0]
    acc = jnp.zeros((blk, hidden), jnp.float32) + b1_ref[...]
    for p in range(seq):
        xp = x3_ref[p]
        parp = par_ref[p].reshape(blk, 1)
        sel = jnp.where(parp > 0.5, xp[:, 64:], xp[:, :64])
        acc += jnp.dot(sel.astype(jnp.bfloat16), w13_ref[p],
                       preferred_element_type=jnp.float32)
    h = jnp.maximum(acc, 0.0)
    logits = jnp.dot(h, w2_ref[...],
                     preferred_element_type=jnp.float32) + b2_ref[...]
    m = jnp.max(logits, axis=1, keepdims=True)
    e = logits - m
    lse = jnp.log(jnp.sum(jnp.exp(e), axis=1, keepdims=True))
    out_ref[...] = e - lse


def _tc_mlp(x3, par, w1, b1, w2, b2, num_tags):
    seq, bs, _ = x3.shape
    in_dim, hidden = w1.shape
    emb = in_dim // seq
    blk = 2048
    grid = bs // blk
    body = functools.partial(_mlp_body, seq=seq, hidden=hidden)
    return pl.pallas_call(
        body,
        grid=(grid,),
        in_specs=[
            pl.BlockSpec((seq, blk, 128), lambda i: (0, i, 0)),
            pl.BlockSpec((seq, blk), lambda i: (0, i)),
            pl.BlockSpec((seq, emb, hidden), lambda i: (0, 0, 0)),
            pl.BlockSpec((1, hidden), lambda i: (0, 0)),
            pl.BlockSpec((hidden, num_tags), lambda i: (0, 0)),
            pl.BlockSpec((1, num_tags), lambda i: (0, 0)),
        ],
        out_specs=pl.BlockSpec((blk, num_tags), lambda i: (i, 0)),
        out_shape=jax.ShapeDtypeStruct((bs, num_tags), jnp.float32),
    )(x3, par, w1.reshape(seq, emb, hidden).astype(jnp.bfloat16),
      b1.reshape(1, hidden), w2, b2.reshape(1, num_tags))


def kernel(Xtoks_IDs, emb_table, W1, b1, W2, b2):
    bs, seq = Xtoks_IDs.shape
    vocab, emb = emb_table.shape
    num_tags = W2.shape[1]

    toks_t = Xtoks_IDs.astype(jnp.int32).T
    blk_i = toks_t // BV
    loc = toks_t % BV
    pair_idx = (blk_i * (BV // 2) + loc % (BV // 2)).reshape(-1)
    par = (loc // (BV // 2)).astype(jnp.float32)

    table2 = _tc_reformat(emb_table.T)
    rows = _sc_gather(pair_idx, table2, bs * seq)
    x3 = rows.reshape(seq, bs, 2 * emb)
    return _tc_mlp(x3, par, W1, b1, W2, b2, num_tags)


# speedup vs baseline: 2.7438x; 1.0213x over previous
import functools

import jax
import jax.numpy as jnp
from jax import lax
from jax.experimental import pallas as pl
from jax.experimental.pallas import tpu as pltpu
from jax.experimental.pallas import tpu_sc as plsc

NC = 2
NS = 16
NW = NC * NS
CH = 128
BV = 32768


def _reformat_body(xt_ref, out_ref):
    x = xt_ref[...]
    z = jnp.concatenate([x[:, :BV // 2], x[:, BV // 2:]], axis=0)
    out_ref[...] = z.T


def _tc_reformat(table_t):
    emb, vocab = table_t.shape
    grid = pl.cdiv(vocab, BV)
    return pl.pallas_call(
        _reformat_body,
        grid=(grid,),
        in_specs=[pl.BlockSpec((emb, BV), lambda i: (0, i))],
        out_specs=pl.BlockSpec((BV // 2, 2 * emb), lambda i: (i, 0)),
        out_shape=jax.ShapeDtypeStruct((grid * BV // 2, 2 * emb), jnp.float32),
    )(table_t)


def _gather_body(idx_hbm, table_hbm, out_hbm, idx_v, buf0, buf1, sem0, sem1,
                 *, n_chunk):
    wid = lax.axis_index("s") * NC + lax.axis_index("c")
    rows_w = n_chunk * CH
    base = wid * rows_w
    pltpu.sync_copy(idx_hbm.at[pl.ds(base, rows_w)], idx_v)

    bufs = (buf0, buf1)
    sems = (sem0, sem1)

    def start(j):
        return pltpu.async_copy(
            table_hbm.at[idx_v.at[pl.ds(j * CH, CH)]], bufs[j % 2], sems[j % 2])

    descs = [None] * n_chunk
    descs[0] = start(0)
    for j in range(n_chunk):
        if j + 1 < n_chunk:
            descs[j + 1] = start(j + 1)
        descs[j].wait()
        pltpu.sync_copy(bufs[j % 2], out_hbm.at[pl.ds(base + j * CH, CH)])


def _sc_gather(idx, table2, n_rows):
    n_chunk = n_rows // (NW * CH)
    mesh = plsc.VectorSubcoreMesh(core_axis_name="c", subcore_axis_name="s")
    body = functools.partial(_gather_body, n_chunk=n_chunk)
    return pl.kernel(
        body,
        out_type=jax.ShapeDtypeStruct((n_rows, 128), jnp.float32),
        mesh=mesh,
        scratch_types=[
            pltpu.VMEM((n_rows // NW,), jnp.int32),
            pltpu.VMEM((CH, 128), jnp.float32),
            pltpu.VMEM((CH, 128), jnp.float32),
            pltpu.SemaphoreType.DMA,
            pltpu.SemaphoreType.DMA,
        ],
    )(idx, table2)


def _mlp_body(x3_ref, par_ref, w13_ref, b1_ref, w2_ref, b2_ref, out_ref,
              *, seq, hidden):
    blk = out_ref.shape[0]
    acc = jnp.zeros((blk, hidden), jnp.float32) + b1_ref[...]
    for p in range(seq):
        xp = x3_ref[p]
        parp = par_ref[p].reshape(blk, 1)
        sel = jnp.where(parp > 0.5, xp[:, 64:], xp[:, :64])
        acc += jnp.dot(sel.astype(jnp.bfloat16), w13_ref[p],
                       preferred_element_type=jnp.float32)
    h = jnp.maximum(acc, 0.0)
    logits = jnp.dot(h, w2_ref[...],
                     preferred_element_type=jnp.float32) + b2_ref[...]
    m = jnp.max(logits, axis=1, keepdims=True)
    e = logits - m
    lse = jnp.log(jnp.sum(jnp.exp(e), axis=1, keepdims=True))
    out_ref[...] = e - lse


def _tc_mlp(x3, par, w1, b1, w2, b2, num_tags):
    seq, bs, _ = x3.shape
    in_dim, hidden = w1.shape
    emb = in_dim // seq
    blk = 2048
    grid = bs // blk
    body = functools.partial(_mlp_body, seq=seq, hidden=hidden)
    return pl.pallas_call(
        body,
        grid=(grid,),
        in_specs=[
            pl.BlockSpec((seq, blk, 128), lambda i: (0, i, 0)),
            pl.BlockSpec((seq, blk), lambda i: (0, i)),
            pl.BlockSpec((seq, emb, hidden), lambda i: (0, 0, 0)),
            pl.BlockSpec((1, hidden), lambda i: (0, 0)),
            pl.BlockSpec((hidden, num_tags), lambda i: (0, 0)),
            pl.BlockSpec((1, num_tags), lambda i: (0, 0)),
        ],
        out_specs=pl.BlockSpec((blk, num_tags), lambda i: (i, 0)),
        out_shape=jax.ShapeDtypeStruct((bs, num_tags), jnp.float32),
    )(x3, par, w1.reshape(seq, emb, hidden).astype(jnp.bfloat16),
      b1.reshape(1, hidden), w2, b2.reshape(1, num_tags))


def kernel(Xtoks_IDs, emb_table, W1, b1, W2, b2):
    bs, seq = Xtoks_IDs.shape
    vocab, emb = emb_table.shape
    num_tags = W2.shape[1]

    toks_t = Xtoks_IDs.astype(jnp.int32).T
    blk_i = toks_t // BV
    loc = toks_t % BV
    pair_idx = (blk_i * (BV // 2) + loc % (BV // 2)).reshape(-1)
    par = (loc // (BV // 2)).astype(jnp.float32)

    table2 = _tc_reformat(emb_table.T)
    rows = _sc_gather(pair_idx, table2, bs * seq)
    x3 = rows.reshape(seq, bs, 2 * emb)
    return _tc_mlp(x3, par, W1, b1, W2, b2, num_tags)


# speedup vs baseline: 3.0211x; 1.1010x over previous
import functools

import jax
import jax.numpy as jnp
from jax import lax
from jax.experimental import pallas as pl
from jax.experimental.pallas import tpu as pltpu
from jax.experimental.pallas import tpu_sc as plsc

NC = 2
NS = 16
NW = NC * NS
CH = 128
BV = 32768


def _reformat_body(xt_ref, out_ref):
    x = xt_ref[...]
    z = jnp.concatenate([x[:, :BV // 2], x[:, BV // 2:]], axis=0)
    zt = z.T
    zi = lax.bitcast_convert_type(zt, jnp.int32)
    zb = lax.shift_right_logical(zi + 0x8000, 16)
    out_ref[...] = zb[:BV // 4] | (zb[BV // 4:] << 16)


def _tc_reformat(table_t):
    emb, vocab = table_t.shape
    grid = pl.cdiv(vocab, BV)
    return pl.pallas_call(
        _reformat_body,
        grid=(grid,),
        in_specs=[pl.BlockSpec((emb, BV), lambda i: (0, i))],
        out_specs=pl.BlockSpec((BV // 4, 2 * emb), lambda i: (i, 0)),
        out_shape=jax.ShapeDtypeStruct((grid * BV // 4, 2 * emb), jnp.int32),
    )(table_t)


def _gather_body(idx_hbm, table_hbm, out_hbm, idx_v, buf0, buf1, sem0, sem1,
                 *, n_chunk):
    wid = lax.axis_index("s") * NC + lax.axis_index("c")
    rows_w = n_chunk * CH
    base = wid * rows_w
    pltpu.sync_copy(idx_hbm.at[pl.ds(base, rows_w)], idx_v)

    bufs = (buf0, buf1)
    sems = (sem0, sem1)

    def start(j):
        return pltpu.async_copy(
            table_hbm.at[idx_v.at[pl.ds(j * CH, CH)]], bufs[j % 2], sems[j % 2])

    descs = [None] * n_chunk
    descs[0] = start(0)
    for j in range(n_chunk):
        if j + 1 < n_chunk:
            descs[j + 1] = start(j + 1)
        descs[j].wait()
        pltpu.sync_copy(bufs[j % 2], out_hbm.at[pl.ds(base + j * CH, CH)])


def _sc_gather(idx, table2, n_rows):
    n_chunk = n_rows // (NW * CH)
    mesh = plsc.VectorSubcoreMesh(core_axis_name="c", subcore_axis_name="s")
    body = functools.partial(_gather_body, n_chunk=n_chunk)
    return pl.kernel(
        body,
        out_type=jax.ShapeDtypeStruct((n_rows, 128), jnp.int32),
        mesh=mesh,
        scratch_types=[
            pltpu.VMEM((n_rows // NW,), jnp.int32),
            pltpu.VMEM((CH, 128), jnp.int32),
            pltpu.VMEM((CH, 128), jnp.int32),
            pltpu.SemaphoreType.DMA,
            pltpu.SemaphoreType.DMA,
        ],
    )(idx, table2)


def _mlp_body(x3_ref, par_ref, sub_ref, w13_ref, b1_ref, w2_ref, b2_ref,
              out_ref, *, seq, hidden):
    blk = out_ref.shape[0]
    acc = jnp.zeros((blk, hidden), jnp.float32) + b1_ref[...]
    for p in range(seq):
        xp = x3_ref[p]
        subp = sub_ref[p].reshape(blk, 1)
        bits = lax.shift_left(xp, (1 - subp) * 16) & jnp.int32(-65536)
        xf = lax.bitcast_convert_type(bits, jnp.float32)
        parp = par_ref[p].reshape(blk, 1)
        sel = jnp.where(parp > 0.5, xf[:, 64:], xf[:, :64])
        acc += jnp.dot(sel.astype(jnp.bfloat16), w13_ref[p],
                       preferred_element_type=jnp.float32)
    h = jnp.maximum(acc, 0.0)
    logits = jnp.dot(h, w2_ref[...],
                     preferred_element_type=jnp.float32) + b2_ref[...]
    m = jnp.max(logits, axis=1, keepdims=True)
    e = logits - m
    lse = jnp.log(jnp.sum(jnp.exp(e), axis=1, keepdims=True))
    out_ref[...] = e - lse


def _tc_mlp(x3, par, sub, w1, b1, w2, b2, num_tags):
    seq, bs, _ = x3.shape
    in_dim, hidden = w1.shape
    emb = in_dim // seq
    blk = 2048
    grid = bs // blk
    body = functools.partial(_mlp_body, seq=seq, hidden=hidden)
    return pl.pallas_call(
        body,
        grid=(grid,),
        in_specs=[
            pl.BlockSpec((seq, blk, 128), lambda i: (0, i, 0)),
            pl.BlockSpec((seq, blk), lambda i: (0, i)),
            pl.BlockSpec((seq, blk), lambda i: (0, i)),
            pl.BlockSpec((seq, emb, hidden), lambda i: (0, 0, 0)),
            pl.BlockSpec((1, hidden), lambda i: (0, 0)),
            pl.BlockSpec((hidden, num_tags), lambda i: (0, 0)),
            pl.BlockSpec((1, num_tags), lambda i: (0, 0)),
        ],
        out_specs=pl.BlockSpec((blk, num_tags), lambda i: (i, 0)),
        out_shape=jax.ShapeDtypeStruct((bs, num_tags), jnp.float32),
    )(x3, par, sub, w1.reshape(seq, emb, hidden).astype(jnp.bfloat16),
      b1.reshape(1, hidden), w2, b2.reshape(1, num_tags))


def kernel(Xtoks_IDs, emb_table, W1, b1, W2, b2):
    bs, seq = Xtoks_IDs.shape
    vocab, emb = emb_table.shape
    num_tags = W2.shape[1]

    toks_t = Xtoks_IDs.astype(jnp.int32).T
    blk_i = toks_t // BV
    loc = toks_t % BV
    l2 = loc % (BV // 2)
    par = (loc // (BV // 2)).astype(jnp.float32)
    sub = l2 // (BV // 4)
    word_idx = (blk_i * (BV // 4) + l2 % (BV // 4)).reshape(-1)

    table2 = _tc_reformat(emb_table.T)
    rows = _sc_gather(word_idx, table2, bs * seq)
    x3 = rows.reshape(seq, bs, 2 * emb)
    return _tc_mlp(x3, par, sub, W1, b1, W2, b2, num_tags)


# speedup vs baseline: 3.0346x; 1.0045x over previous
import functools

import jax
import jax.numpy as jnp
from jax import lax
from jax.experimental import pallas as pl
from jax.experimental.pallas import tpu as pltpu
from jax.experimental.pallas import tpu_sc as plsc

NC = 2
NS = 16
NW = NC * NS
CH = 128
BV = 32768
SUB_SHIFT = (BV // 4).bit_length() - 1
PAR_SHIFT = (BV // 2).bit_length() - 1


def _reformat_body(xt_ref, out_ref):
    x = xt_ref[...]
    z = jnp.concatenate([x[:, :BV // 2], x[:, BV // 2:]], axis=0)
    zt = z.T
    zi = lax.bitcast_convert_type(zt, jnp.int32)
    zb = lax.shift_right_logical(
        zi + 0x7FFF + (lax.shift_right_logical(zi, 16) & 1), 16)
    out_ref[...] = zb[:BV // 4] | (zb[BV // 4:] << 16)


def _tc_reformat(table_t):
    emb, vocab = table_t.shape
    grid = pl.cdiv(vocab, BV)
    return pl.pallas_call(
        _reformat_body,
        grid=(grid,),
        in_specs=[pl.BlockSpec((emb, BV), lambda i: (0, i))],
        out_specs=pl.BlockSpec((BV // 4, 2 * emb), lambda i: (i, 0)),
        out_shape=jax.ShapeDtypeStruct((grid * BV // 4, 2 * emb), jnp.int32),
    )(table_t)


def _gather_body(idx_hbm, table_hbm, out_hbm, idx_v, buf0, buf1, sem0, sem1,
                 *, n_chunk):
    wid = lax.axis_index("s") * NC + lax.axis_index("c")
    rows_w = n_chunk * CH
    base = wid * rows_w
    pltpu.sync_copy(idx_hbm.at[pl.ds(base, rows_w)], idx_v)

    bufs = (buf0, buf1)
    sems = (sem0, sem1)

    def start(j):
        return pltpu.async_copy(
            table_hbm.at[idx_v.at[pl.ds(j * CH, CH)]], bufs[j % 2], sems[j % 2])

    descs = [None] * n_chunk
    descs[0] = start(0)
    for j in range(n_chunk):
        if j + 1 < n_chunk:
            descs[j + 1] = start(j + 1)
        descs[j].wait()
        pltpu.sync_copy(bufs[j % 2], out_hbm.at[pl.ds(base + j * CH, CH)])


def _sc_gather(idx, table2, n_rows):
    n_chunk = n_rows // (NW * CH)
    mesh = plsc.VectorSubcoreMesh(core_axis_name="c", subcore_axis_name="s")
    body = functools.partial(_gather_body, n_chunk=n_chunk)
    return pl.kernel(
        body,
        out_type=jax.ShapeDtypeStruct((n_rows, 128), jnp.int32),
        mesh=mesh,
        scratch_types=[
            pltpu.VMEM((n_rows // NW,), jnp.int32),
            pltpu.VMEM((CH, 128), jnp.int32),
            pltpu.VMEM((CH, 128), jnp.int32),
            pltpu.SemaphoreType.DMA,
            pltpu.SemaphoreType.DMA,
        ],
    )(idx, table2)


def _mlp_body(x3_ref, toks_ref, w13_ref, b1_ref, w2_ref, b2_ref,
              out_ref, *, seq, hidden):
    blk = out_ref.shape[0]
    acc = jnp.zeros((blk, hidden), jnp.float32) + b1_ref[...]
    for p in range(seq):
        xp = x3_ref[p]
        tok = toks_ref[:, p].reshape(blk, 1)
        subp = lax.shift_right_logical(tok, SUB_SHIFT) & 1
        bits = lax.shift_left(xp, (1 - subp) * 16) & jnp.int32(-65536)
        xf = lax.bitcast_convert_type(bits, jnp.float32)
        parp = lax.shift_right_logical(tok, PAR_SHIFT) & 1
        sel = jnp.where(parp > 0, xf[:, 64:], xf[:, :64])
        acc += jnp.dot(sel.astype(jnp.bfloat16), w13_ref[p],
                       preferred_element_type=jnp.float32)
    h = jnp.maximum(acc, 0.0)
    logits = jnp.dot(h, w2_ref[...],
                     preferred_element_type=jnp.float32) + b2_ref[...]
    m = jnp.max(logits, axis=1, keepdims=True)
    e = logits - m
    lse = jnp.log(jnp.sum(jnp.exp(e), axis=1, keepdims=True))
    out_ref[...] = e - lse


def _tc_mlp(x3, toks, w1, b1, w2, b2, num_tags):
    seq, bs, _ = x3.shape
    in_dim, hidden = w1.shape
    emb = in_dim // seq
    blk = 2048
    grid = bs // blk
    body = functools.partial(_mlp_body, seq=seq, hidden=hidden)
    return pl.pallas_call(
        body,
        grid=(grid,),
        in_specs=[
            pl.BlockSpec((seq, blk, 128), lambda i: (0, i, 0)),
            pl.BlockSpec((blk, seq), lambda i: (i, 0)),
            pl.BlockSpec((seq, emb, hidden), lambda i: (0, 0, 0)),
            pl.BlockSpec((1, hidden), lambda i: (0, 0)),
            pl.BlockSpec((hidden, num_tags), lambda i: (0, 0)),
            pl.BlockSpec((1, num_tags), lambda i: (0, 0)),
        ],
        out_specs=pl.BlockSpec((blk, num_tags), lambda i: (i, 0)),
        out_shape=jax.ShapeDtypeStruct((bs, num_tags), jnp.float32),
    )(x3, toks, w1.reshape(seq, emb, hidden).astype(jnp.bfloat16),
      b1.reshape(1, hidden), w2, b2.reshape(1, num_tags))


def kernel(Xtoks_IDs, emb_table, W1, b1, W2, b2):
    bs, seq = Xtoks_IDs.shape
    vocab, emb = emb_table.shape
    num_tags = W2.shape[1]

    toks_t = Xtoks_IDs.astype(jnp.int32).T
    blk_i = toks_t // BV
    l2 = (toks_t % BV) % (BV // 2)
    word_idx = (blk_i * (BV // 4) + l2 % (BV // 4)).reshape(-1)

    table2 = _tc_reformat(emb_table.T)
    rows = _sc_gather(word_idx, table2, bs * seq)
    x3 = rows.reshape(seq, bs, 2 * emb)
    return _tc_mlp(x3, Xtoks_IDs.astype(jnp.int32), W1, b1, W2, b2,
                   num_tags)
